# baseline (speedup 1.0000x reference)
import jax
import jax.numpy as jnp
from jax import lax
from jax.experimental import pallas as pl
from jax.experimental.pallas import tpu as pltpu

_PB = 16


def _row_log_softmax(x):
    m = jnp.max(x, axis=-1, keepdims=True)
    s = x - m
    lse = jnp.log(jnp.sum(jnp.exp(s), axis=-1, keepdims=True))
    return s - lse


def _kl_partial(dec_blk, sub_blk):
    logp_x = _row_log_softmax(dec_blk)
    logp_y = _row_log_softmax(sub_blk)
    p_y = jnp.exp(logp_y)
    return jnp.sum(p_y * (logp_y - logp_x))


def _fused_body(cell_ref, gene_full_ref, gene_blk_ref, gcs_ref, pcs_ref,
                w_ref, b_ref, peak_blk_ref, out_ref, loss_ref, gp_ref,
                acc_ref):
    i = pl.program_id(0)
    n = pl.num_programs(0)
    h = cell_ref.shape[1]
    c = cell_ref.shape[0]
    g = gene_full_ref.shape[0]
    p = n * _PB

    @pl.when(i == 0)
    def _():
        gp_ref[...] = lax.dot_general(
            gene_full_ref[...], w_ref[:, :h], (((1,), (1,)), ((), ())),
            preferred_element_type=jnp.float32)
        acc_ref[...] = jnp.zeros((1, 1), jnp.float32)

    pp = lax.dot_general(peak_blk_ref[...], w_ref[:, h:],
                         (((1,), (1,)), ((), ())),
                         preferred_element_type=jnp.float32) + b_ref[...]
    out = jnp.maximum(gp_ref[...][None, :, :] + pp[:, None, :], 0.0)
    out_ref[...] = jnp.reshape(out, (_PB * g, h))

    cell = cell_ref[...]
    dec1 = lax.dot_general(gene_blk_ref[...], cell, (((1,), (1,)), ((), ())),
                           preferred_element_type=jnp.float32)
    dec2 = lax.dot_general(peak_blk_ref[...], cell, (((1,), (1,)), ((), ())),
                           preferred_element_type=jnp.float32)
    l1 = _kl_partial(dec1, gcs_ref[...]) / (g * c)
    l2 = _kl_partial(dec2, pcs_ref[...]) / (p * c)
    acc_ref[...] += jnp.reshape(l1 + l2, (1, 1))

    @pl.when(i == n - 1)
    def _():
        loss_ref[...] = acc_ref[...]


def kernel(cell_emb, gene_emb, peak_emb, gene_cell_sub, peak_cell_sub, W, b):
    c, h = cell_emb.shape
    g = gene_emb.shape[0]
    p = peak_emb.shape[0]
    grid = p // _PB
    gb = g // grid
    b2d = jnp.reshape(b, (1, h))

    full = lambda shape: pl.BlockSpec(shape, lambda i: (0, 0))
    out, loss = pl.pallas_call(
        _fused_body,
        grid=(grid,),
        in_specs=[
            full((c, h)),
            full((g, h)),
            pl.BlockSpec((gb, h), lambda i: (i, 0)),
            pl.BlockSpec((gb, c), lambda i: (i, 0)),
            pl.BlockSpec((_PB, c), lambda i: (i, 0)),
            full((h, 2 * h)),
            full((1, h)),
            pl.BlockSpec((_PB, h), lambda i: (i, 0)),
        ],
        out_specs=[
            pl.BlockSpec((_PB * g, h), lambda i: (i, 0)),
            pl.BlockSpec((1, 1), lambda i: (0, 0)),
        ],
        out_shape=[
            jax.ShapeDtypeStruct((p * g, h), jnp.float32),
            jax.ShapeDtypeStruct((1, 1), jnp.float32),
        ],
        scratch_shapes=[
            pltpu.VMEM((g, h), jnp.float32),
            pltpu.VMEM((1, 1), jnp.float32),
        ],
    )(cell_emb, gene_emb, gene_emb, gene_cell_sub, peak_cell_sub, W, b2d,
      peak_emb)
    return out, jnp.reshape(loss, ())


# speedup vs baseline: 1.0438x; 1.0438x over previous
import jax
import jax.numpy as jnp
from jax import lax
from jax.experimental import pallas as pl
from jax.experimental.pallas import tpu as pltpu

_PB = 16


def _row_log_softmax(x):
    m = jnp.max(x, axis=-1, keepdims=True)
    s = x - m
    lse = jnp.log(jnp.sum(jnp.exp(s), axis=-1, keepdims=True))
    return s - lse


def _kl_partial(dec_blk, sub_blk):
    logp_x = _row_log_softmax(dec_blk)
    logp_y = _row_log_softmax(sub_blk)
    p_y = jnp.exp(logp_y)
    return jnp.sum(p_y * (logp_y - logp_x))


def _fused_body(cell_ref, gene_full_ref, gene_blk_ref, gcs_ref, pcs_ref,
                w_ref, b_ref, peak_blk_ref, out_ref, loss_ref, gp_ref,
                acc_ref):
    i = pl.program_id(0)
    n = pl.num_programs(0)
    h = cell_ref.shape[1]
    c = cell_ref.shape[0]
    g = gene_full_ref.shape[0]
    p = n * _PB

    @pl.when(i == 0)
    def _():
        gp_ref[...] = lax.dot_general(
            gene_full_ref[...], w_ref[:, :h], (((1,), (1,)), ((), ())),
            preferred_element_type=jnp.float32)
        acc_ref[...] = jnp.zeros((1, 1), jnp.float32)

    pp = lax.dot_general(peak_blk_ref[...], w_ref[:, h:],
                         (((1,), (1,)), ((), ())),
                         preferred_element_type=jnp.float32) + b_ref[...]
    gp = gp_ref[...]
    for k in range(_PB):
        out_ref[pl.ds(k * g, g), :] = jnp.maximum(gp + pp[k:k + 1, :], 0.0)

    cell = cell_ref[...]
    dec1 = lax.dot_general(gene_blk_ref[...], cell, (((1,), (1,)), ((), ())),
                           preferred_element_type=jnp.float32)
    dec2 = lax.dot_general(peak_blk_ref[...], cell, (((1,), (1,)), ((), ())),
                           preferred_element_type=jnp.float32)
    l1 = _kl_partial(dec1, gcs_ref[...]) / (g * c)
    l2 = _kl_partial(dec2, pcs_ref[...]) / (p * c)
    acc_ref[...] += jnp.reshape(l1 + l2, (1, 1))

    @pl.when(i == n - 1)
    def _():
        loss_ref[...] = acc_ref[...]


def kernel(cell_emb, gene_emb, peak_emb, gene_cell_sub, peak_cell_sub, W, b):
    c, h = cell_emb.shape
    g = gene_emb.shape[0]
    p = peak_emb.shape[0]
    grid = p // _PB
    gb = g // grid
    b2d = jnp.reshape(b, (1, h))

    full = lambda shape: pl.BlockSpec(shape, lambda i: (0, 0))
    out, loss = pl.pallas_call(
        _fused_body,
        grid=(grid,),
        in_specs=[
            full((c, h)),
            full((g, h)),
            pl.BlockSpec((gb, h), lambda i: (i, 0)),
            pl.BlockSpec((gb, c), lambda i: (i, 0)),
            pl.BlockSpec((_PB, c), lambda i: (i, 0)),
            full((h, 2 * h)),
            full((1, h)),
            pl.BlockSpec((_PB, h), lambda i: (i, 0)),
        ],
        out_specs=[
            pl.BlockSpec((_PB * g, h), lambda i: (i, 0)),
            pl.BlockSpec((1, 1), lambda i: (0, 0)),
        ],
        out_shape=[
            jax.ShapeDtypeStruct((p * g, h), jnp.float32),
            jax.ShapeDtypeStruct((1, 1), jnp.float32),
        ],
        scratch_shapes=[
            pltpu.VMEM((g, h), jnp.float32),
            pltpu.VMEM((1, 1), jnp.float32),
        ],
    )(cell_emb, gene_emb, gene_emb, gene_cell_sub, peak_cell_sub, W, b2d,
      peak_emb)
    return out, jnp.reshape(loss, ())


# speedup vs baseline: 1.0994x; 1.0533x over previous
import jax
import jax.numpy as jnp
from jax import lax
from jax.experimental import pallas as pl
from jax.experimental.pallas import tpu as pltpu

_PB = 16


def _row_log_softmax(x):
    m = jnp.max(x, axis=-1, keepdims=True)
    s = x - m
    lse = jnp.log(jnp.sum(jnp.exp(s), axis=-1, keepdims=True))
    return s - lse


def _fused_body(cell_ref, gene_ref, peak_full_ref, gcs_ref, pcs_ref, w_ref,
                b_ref, peak_blk_ref, out_ref, loss_ref, gp_ref):
    i = pl.program_id(0)
    h = gene_ref.shape[1]
    g = gene_ref.shape[0]

    @pl.when(i == 0)
    def _():
        gene = gene_ref[...]
        gp_ref[...] = lax.dot_general(
            gene, w_ref[:, :h], (((1,), (1,)), ((), ())),
            preferred_element_type=jnp.float32)
        cell = cell_ref[...]
        dec1 = lax.dot_general(gene, cell, (((1,), (1,)), ((), ())),
                               preferred_element_type=jnp.float32)
        dec2 = lax.dot_general(peak_full_ref[...], cell,
                               (((1,), (1,)), ((), ())),
                               preferred_element_type=jnp.float32)
        logp_x1 = _row_log_softmax(dec1)
        logp_x2 = _row_log_softmax(dec2)
        logp_y1 = _row_log_softmax(gcs_ref[...])
        logp_y2 = _row_log_softmax(pcs_ref[...])
        p_y1 = jnp.exp(logp_y1)
        p_y2 = jnp.exp(logp_y2)
        l1 = jnp.sum(p_y1 * (logp_y1 - logp_x1)) / (dec1.shape[0] * dec1.shape[1])
        l2 = jnp.sum(p_y2 * (logp_y2 - logp_x2)) / (dec2.shape[0] * dec2.shape[1])
        loss_ref[...] = jnp.reshape(l1 + l2, (1, 1))

    pp = lax.dot_general(peak_blk_ref[...], w_ref[:, h:],
                         (((1,), (1,)), ((), ())),
                         preferred_element_type=jnp.float32) + b_ref[...]
    gp = gp_ref[...]
    for k in range(_PB):
        out_ref[pl.ds(k * g, g), :] = jnp.maximum(gp + pp[k:k + 1, :], 0.0)


def kernel(cell_emb, gene_emb, peak_emb, gene_cell_sub, peak_cell_sub, W, b):
    c, h = cell_emb.shape
    g = gene_emb.shape[0]
    p = peak_emb.shape[0]
    grid = p // _PB
    b2d = jnp.reshape(b, (1, h))

    full = lambda shape: pl.BlockSpec(shape, lambda i: (0, 0))
    out, loss = pl.pallas_call(
        _fused_body,
        grid=(grid,),
        in_specs=[
            full((c, h)),
            full((g, h)),
            full((p, h)),
            full((g, c)),
            full((p, c)),
            full((h, 2 * h)),
            full((1, h)),
            pl.BlockSpec((_PB, h), lambda i: (i, 0)),
        ],
        out_specs=[
            pl.BlockSpec((_PB * g, h), lambda i: (i, 0)),
            pl.BlockSpec((1, 1), lambda i: (0, 0)),
        ],
        out_shape=[
            jax.ShapeDtypeStruct((p * g, h), jnp.float32),
            jax.ShapeDtypeStruct((1, 1), jnp.float32),
        ],
        scratch_shapes=[pltpu.VMEM((g, h), jnp.float32)],
    )(cell_emb, gene_emb, peak_emb, gene_cell_sub, peak_cell_sub, W, b2d,
      peak_emb)
    return out, jnp.reshape(loss, ())
